# Initial kernel scaffold; baseline (speedup 1.0000x reference)
import jax
import jax.numpy as jnp
from jax import lax
from jax.experimental import pallas as pl
from jax.experimental.pallas import tpu as pltpu
from jax.experimental.pallas import tpu_sc as plsc

N_NODES = 100000
N_EDGES = 3200000
D = 16
B = 125
K = 16
C = K * B
N_SUBCORES = 16
EDGES_PER_TILE = N_EDGES // N_SUBCORES
CHUNKS = EDGES_PER_TILE // C
IDX_ROWS = N_EDGES // B
IDX_ROWS_PER_TILE = IDX_ROWS // N_SUBCORES
NODES_PER_TILE = N_NODES // N_SUBCORES


def _body(emb_hbm, src_hbm, dst_hbm, hout_hbm, hin_hbm, rows_v, idx_v, acc):
    cid = lax.axis_index("c")
    sid = lax.axis_index("s")

    def _zero(i, _):
        rows_v[i] = jnp.zeros((D,), jnp.float32)
        return 0

    lax.fori_loop(0, C, _zero, 0)
    n0 = sid * NODES_PER_TILE
    pltpu.sync_copy(rows_v, acc.at[pl.ds(n0, C)])
    pltpu.sync_copy(rows_v, acc.at[pl.ds(n0 + C, C)])
    pltpu.sync_copy(rows_v, acc.at[pl.ds(n0 + 2 * C, C)])
    pltpu.sync_copy(rows_v.at[pl.ds(0, NODES_PER_TILE - 3 * C)],
                    acc.at[pl.ds(n0 + 3 * C, NODES_PER_TILE - 3 * C)])
    plsc.subcore_barrier()

    e_base = sid * EDGES_PER_TILE
    i_base = sid * IDX_ROWS_PER_TILE

    def _chunk(ci, _):
        pltpu.sync_copy(emb_hbm.at[pl.ds(e_base + ci * C, C)], rows_v)

        @pl.when(cid == 0)
        def _():
            pltpu.sync_copy(src_hbm.at[pl.ds(i_base + ci * K, K)], idx_v)

        @pl.when(cid == 1)
        def _():
            pltpu.sync_copy(dst_hbm.at[pl.ds(i_base + ci * K, K)], idx_v)

        for j in range(K):
            pltpu.sync_copy(rows_v.at[pl.ds(j * B, B)],
                            acc.at[idx_v.at[j]], add=True)
        return 0

    lax.fori_loop(0, CHUNKS, _chunk, 0)
    plsc.subcore_barrier()

    @pl.when(cid == 0)
    def _():
        pltpu.sync_copy(acc.at[pl.ds(n0, NODES_PER_TILE)],
                        hout_hbm.at[pl.ds(n0, NODES_PER_TILE)])

    @pl.when(cid == 1)
    def _():
        pltpu.sync_copy(acc.at[pl.ds(n0, NODES_PER_TILE)],
                        hin_hbm.at[pl.ds(n0, NODES_PER_TILE)])


@jax.jit
def _segment_sums(edge_embedding, src_idx, dst_idx):
    mesh = plsc.VectorSubcoreMesh(core_axis_name="c", subcore_axis_name="s")
    f = pl.kernel(
        _body,
        out_type=[
            jax.ShapeDtypeStruct((N_NODES, D), jnp.float32),
            jax.ShapeDtypeStruct((N_NODES, D), jnp.float32),
        ],
        mesh=mesh,
        scratch_types=[
            pltpu.VMEM((C, D), jnp.float32),
            pltpu.VMEM((K, B), jnp.int32),
            pltpu.VMEM_SHARED((N_NODES, D), jnp.float32),
        ],
    )
    return f(edge_embedding, src_idx, dst_idx)


def kernel(edge_embedding, edge_index, coords):
    src_idx = edge_index[0].reshape(IDX_ROWS, B)
    dst_idx = edge_index[1].reshape(IDX_ROWS, B)
    h_out, h_in = _segment_sums(edge_embedding, src_idx, dst_idx)
    return jnp.concatenate([h_out, h_in, coords], axis=1)


# speedup vs baseline: 8.0622x; 8.0622x over previous
import jax
import jax.numpy as jnp
from jax import lax
from jax.experimental import pallas as pl
from jax.experimental.pallas import tpu as pltpu
from jax.experimental.pallas import tpu_sc as plsc

N_NODES = 100000
N_EDGES = 3200000
D = 16
B = 100
K = 16
C = K * B
N_SUBCORES = 16
EDGES_PER_TILE = N_EDGES // N_SUBCORES
CHUNKS = EDGES_PER_TILE // C
IDX_ROWS = N_EDGES // B
IDX_ROWS_PER_TILE = IDX_ROWS // N_SUBCORES
NODES_PER_TILE = 6256
NODES_LAST_TILE = N_NODES - 15 * NODES_PER_TILE


def _body(emb_hbm, src_hbm, dst_hbm, hout_hbm, hin_hbm, rows_v, idx_v, acc):
    cid = lax.axis_index("c")
    sid = lax.axis_index("s")

    def _zero(i, _):
        rows_v[i] = jnp.zeros((D,), jnp.float32)
        return 0

    lax.fori_loop(0, C, _zero, 0)
    n0 = sid * NODES_PER_TILE
    for k in range(NODES_PER_TILE // C):
        pltpu.sync_copy(rows_v, acc.at[pl.ds(n0 + k * C, C)])
    _full = (NODES_PER_TILE // C) * C

    @pl.when(sid < 15)
    def _():
        pltpu.sync_copy(rows_v.at[pl.ds(0, NODES_PER_TILE - _full)],
                        acc.at[pl.ds(n0 + _full, NODES_PER_TILE - _full)])

    @pl.when(sid == 15)
    def _():
        pltpu.sync_copy(rows_v.at[pl.ds(0, NODES_LAST_TILE - _full)],
                        acc.at[pl.ds(n0 + _full, NODES_LAST_TILE - _full)])

    plsc.subcore_barrier()

    e_base = sid * EDGES_PER_TILE
    i_base = sid * IDX_ROWS_PER_TILE

    def _chunk(ci, _):
        pltpu.sync_copy(emb_hbm.at[pl.ds(e_base + ci * C, C)], rows_v)

        @pl.when(cid == 0)
        def _():
            pltpu.sync_copy(src_hbm.at[pl.ds(i_base + ci * K, K)], idx_v)

        @pl.when(cid == 1)
        def _():
            pltpu.sync_copy(dst_hbm.at[pl.ds(i_base + ci * K, K)], idx_v)

        for j in range(K):
            pltpu.sync_copy(rows_v.at[pl.ds(j * B, B)],
                            acc.at[idx_v.at[j]], add=True)
        return 0

    lax.fori_loop(0, CHUNKS, _chunk, 0)
    plsc.subcore_barrier()

    out_hbm_sel = [hout_hbm, hin_hbm]
    for core in (0, 1):
        @pl.when(cid == core)
        def _(out=out_hbm_sel[core]):
            @pl.when(sid < 15)
            def _():
                pltpu.sync_copy(acc.at[pl.ds(n0, NODES_PER_TILE)],
                                out.at[pl.ds(n0, NODES_PER_TILE)])

            @pl.when(sid == 15)
            def _():
                pltpu.sync_copy(acc.at[pl.ds(n0, NODES_LAST_TILE)],
                                out.at[pl.ds(n0, NODES_LAST_TILE)])


@jax.jit
def _segment_sums(edge_embedding, src_idx, dst_idx):
    mesh = plsc.VectorSubcoreMesh(core_axis_name="c", subcore_axis_name="s")
    f = pl.kernel(
        _body,
        out_type=[
            jax.ShapeDtypeStruct((N_NODES, D), jnp.float32),
            jax.ShapeDtypeStruct((N_NODES, D), jnp.float32),
        ],
        mesh=mesh,
        scratch_types=[
            pltpu.VMEM((C, D), jnp.float32),
            pltpu.VMEM((K, B), jnp.int32),
            pltpu.VMEM_SHARED((N_NODES, D), jnp.float32),
        ],
        compiler_params=pltpu.CompilerParams(use_tc_tiling_on_sc=False),
    )
    return f(edge_embedding, src_idx, dst_idx)


def kernel(edge_embedding, edge_index, coords):
    src_idx = edge_index[0].reshape(IDX_ROWS, B)
    dst_idx = edge_index[1].reshape(IDX_ROWS, B)
    h_out, h_in = _segment_sums(edge_embedding, src_idx, dst_idx)
    return jnp.concatenate([h_out, h_in, coords], axis=1)


# speedup vs baseline: 8.5689x; 1.0629x over previous
import jax
import jax.numpy as jnp
from jax import lax
from jax.experimental import pallas as pl
from jax.experimental.pallas import tpu as pltpu
from jax.experimental.pallas import tpu_sc as plsc

N_NODES = 100000
N_EDGES = 3200000
D = 16
B = 100
K = 16
C = K * B
N_SUBCORES = 16
EDGES_PER_TILE = N_EDGES // N_SUBCORES
CHUNKS = EDGES_PER_TILE // C
IDX_ROWS = N_EDGES // B
IDX_ROWS_PER_TILE = IDX_ROWS // N_SUBCORES
NODES_PER_TILE = 6256
NODES_LAST_TILE = N_NODES - 15 * NODES_PER_TILE


def _body(emb_hbm, src_hbm, dst_hbm, hout_hbm, hin_hbm, rows_v, idx_v, acc,
          sem):
    cid = lax.axis_index("c")
    sid = lax.axis_index("s")

    def _zero(i, _):
        rows_v[i] = jnp.zeros((D,), jnp.float32)
        return 0

    lax.fori_loop(0, C, _zero, 0)
    n0 = sid * NODES_PER_TILE
    for k in range(NODES_PER_TILE // C):
        pltpu.sync_copy(rows_v, acc.at[pl.ds(n0 + k * C, C)])
    _full = (NODES_PER_TILE // C) * C

    @pl.when(sid < 15)
    def _():
        pltpu.sync_copy(rows_v.at[pl.ds(0, NODES_PER_TILE - _full)],
                        acc.at[pl.ds(n0 + _full, NODES_PER_TILE - _full)])

    @pl.when(sid == 15)
    def _():
        pltpu.sync_copy(rows_v.at[pl.ds(0, NODES_LAST_TILE - _full)],
                        acc.at[pl.ds(n0 + _full, NODES_LAST_TILE - _full)])

    plsc.subcore_barrier()

    e_base = sid * EDGES_PER_TILE
    i_base = sid * IDX_ROWS_PER_TILE

    def _chunk(ci, _):
        pltpu.sync_copy(emb_hbm.at[pl.ds(e_base + ci * C, C)], rows_v)

        @pl.when(cid == 0)
        def _():
            pltpu.sync_copy(src_hbm.at[pl.ds(i_base + ci * K, K)], idx_v)

        @pl.when(cid == 1)
        def _():
            pltpu.sync_copy(dst_hbm.at[pl.ds(i_base + ci * K, K)], idx_v)

        descs = [
            pltpu.async_copy(rows_v.at[pl.ds(j * B, B)],
                             acc.at[idx_v.at[j]], sem, add=True)
            for j in range(K)
        ]
        for d in descs:
            d.wait()
        return 0

    lax.fori_loop(0, CHUNKS, _chunk, 0)
    plsc.subcore_barrier()

    out_hbm_sel = [hout_hbm, hin_hbm]
    for core in (0, 1):
        @pl.when(cid == core)
        def _(out=out_hbm_sel[core]):
            @pl.when(sid < 15)
            def _():
                pltpu.sync_copy(acc.at[pl.ds(n0, NODES_PER_TILE)],
                                out.at[pl.ds(n0, NODES_PER_TILE)])

            @pl.when(sid == 15)
            def _():
                pltpu.sync_copy(acc.at[pl.ds(n0, NODES_LAST_TILE)],
                                out.at[pl.ds(n0, NODES_LAST_TILE)])


@jax.jit
def _segment_sums(edge_embedding, src_idx, dst_idx):
    mesh = plsc.VectorSubcoreMesh(core_axis_name="c", subcore_axis_name="s")
    f = pl.kernel(
        _body,
        out_type=[
            jax.ShapeDtypeStruct((N_NODES, D), jnp.float32),
            jax.ShapeDtypeStruct((N_NODES, D), jnp.float32),
        ],
        mesh=mesh,
        scratch_types=[
            pltpu.VMEM((C, D), jnp.float32),
            pltpu.VMEM((K, B), jnp.int32),
            pltpu.VMEM_SHARED((N_NODES, D), jnp.float32),
            pltpu.SemaphoreType.DMA,
        ],
        compiler_params=pltpu.CompilerParams(use_tc_tiling_on_sc=False),
    )
    return f(edge_embedding, src_idx, dst_idx)


def kernel(edge_embedding, edge_index, coords):
    src_idx = edge_index[0].reshape(IDX_ROWS, B)
    dst_idx = edge_index[1].reshape(IDX_ROWS, B)
    h_out, h_in = _segment_sums(edge_embedding, src_idx, dst_idx)
    return jnp.concatenate([h_out, h_in, coords], axis=1)


# speedup vs baseline: 8.6678x; 1.0115x over previous
import jax
import jax.numpy as jnp
from jax import lax
from jax.experimental import pallas as pl
from jax.experimental.pallas import tpu as pltpu
from jax.experimental.pallas import tpu_sc as plsc

N_NODES = 100000
N_EDGES = 3200000
D = 16
B = 128
K = 8
C = K * B
N_SUBCORES = 16
IDX_ROWS = N_EDGES // B
IR_PER_TILE = 1568
IR_LAST_TILE = IDX_ROWS - 15 * IR_PER_TILE
CHUNKS = IR_PER_TILE // K
CHUNKS_LAST = IR_LAST_TILE // K
NODES_PER_TILE = 6256
NODES_LAST_TILE = N_NODES - 15 * NODES_PER_TILE


def _body(emb_hbm, eidx_hbm, hout_hbm, hin_hbm, rows_v, idx_v, acc, sem):
    cid = lax.axis_index("c")
    sid = lax.axis_index("s")

    def _zero(i, _):
        rows_v[i] = jnp.zeros((D,), jnp.float32)
        return 0

    lax.fori_loop(0, C, _zero, 0)
    n0 = sid * NODES_PER_TILE
    for k in range(NODES_PER_TILE // C):
        pltpu.sync_copy(rows_v, acc.at[pl.ds(n0 + k * C, C)])
    _full = (NODES_PER_TILE // C) * C

    @pl.when(sid < 15)
    def _():
        pltpu.sync_copy(rows_v.at[pl.ds(0, NODES_PER_TILE - _full)],
                        acc.at[pl.ds(n0 + _full, NODES_PER_TILE - _full)])

    @pl.when(sid == 15)
    def _():
        pltpu.sync_copy(rows_v.at[pl.ds(0, NODES_LAST_TILE - _full)],
                        acc.at[pl.ds(n0 + _full, NODES_LAST_TILE - _full)])

    plsc.subcore_barrier()

    i_base = sid * IR_PER_TILE
    e_base = i_base * B
    n_chunks = jnp.where(sid < 15, CHUNKS, CHUNKS_LAST)

    def _chunk(ci, _):
        pltpu.sync_copy(emb_hbm.at[pl.ds(e_base + ci * C, C)], rows_v)
        pltpu.sync_copy(eidx_hbm.at[cid, pl.ds(i_base + ci * K, K)], idx_v)
        descs = [
            pltpu.async_copy(rows_v.at[pl.ds(j * B, B)],
                             acc.at[idx_v.at[j]], sem, add=True)
            for j in range(K)
        ]
        for d in descs:
            d.wait()
        return 0

    lax.fori_loop(0, n_chunks, _chunk, 0)
    plsc.subcore_barrier()

    out_hbm_sel = [hout_hbm, hin_hbm]
    for core in (0, 1):
        @pl.when(cid == core)
        def _(out=out_hbm_sel[core]):
            @pl.when(sid < 15)
            def _():
                pltpu.sync_copy(acc.at[pl.ds(n0, NODES_PER_TILE)],
                                out.at[pl.ds(n0, NODES_PER_TILE)])

            @pl.when(sid == 15)
            def _():
                pltpu.sync_copy(acc.at[pl.ds(n0, NODES_LAST_TILE)],
                                out.at[pl.ds(n0, NODES_LAST_TILE)])


@jax.jit
def _segment_sums(edge_embedding, eidx3):
    mesh = plsc.VectorSubcoreMesh(core_axis_name="c", subcore_axis_name="s")
    f = pl.kernel(
        _body,
        out_type=[
            jax.ShapeDtypeStruct((N_NODES, D), jnp.float32),
            jax.ShapeDtypeStruct((N_NODES, D), jnp.float32),
        ],
        mesh=mesh,
        scratch_types=[
            pltpu.VMEM((C, D), jnp.float32),
            pltpu.VMEM((K, B), jnp.int32),
            pltpu.VMEM_SHARED((N_NODES, D), jnp.float32),
            pltpu.SemaphoreType.DMA,
        ],
        compiler_params=pltpu.CompilerParams(use_tc_tiling_on_sc=False),
    )
    return f(edge_embedding, eidx3)


def kernel(edge_embedding, edge_index, coords):
    eidx3 = edge_index.reshape(2, IDX_ROWS, B)
    h_out, h_in = _segment_sums(edge_embedding, eidx3)
    return jnp.concatenate([h_out, h_in, coords], axis=1)
